# Initial kernel scaffold; baseline (speedup 1.0000x reference)
import jax
import jax.numpy as jnp
from jax.experimental import pallas as pl


def kernel(mem, srcID, src_feature, dstID, dst_feature):
    raise NotImplementedError("write your pallas kernel here")


# speedup vs baseline: 2.1891x; 2.1891x over previous
import functools

import jax
import jax.numpy as jnp
from jax import lax
from jax.experimental import pallas as pl
from jax.experimental.pallas import tpu as pltpu
from jax.experimental.pallas import tpu_sc as plsc

_M = 1_000_000
_D = 15
_B = 16_384
_NB = 2 * _B
_NC = 2
_NS = 16
_NW = _NC * _NS
_NG = _M // 8
_CR = 1024
_LK = 128
_GROWS = _NB // _LK + 2
_PCAP = 2048
_PROWS = _PCAP // _LK
_SCAN_CHUNK = 4096
_NSCAN = _NB // _SCAN_CHUNK


def _body(mem, idx, upd, out, idxb, gpk, cpk, cpos, pay, buf, s_g):
    wid = lax.axis_index("c") * _NS + lax.axis_index("s")
    lo = ((wid * _NG) // _NW) * 8
    hi = (((wid + 1) * _NG) // _NW) * 8
    rpw = hi - lo
    iot = lax.iota(jnp.int32, 16)

    def memset_body(i, z):
        cpos[i >> 3, pl.ds((i & 7) * 16, 16)] = jnp.zeros((16,), jnp.int32)
        return z
    lax.fori_loop(0, _PROWS * 8, memset_body, 0)

    def scan_chunk(c, cnt_v):
        pltpu.sync_copy(idx.at[pl.ds(c * _SCAN_CHUNK, _SCAN_CHUNK)], idxb)

        def it(i, cnt_v):
            v = idxb[pl.ds(i * 16, 16)]
            m = (v >= lo) & (v < hi)
            pc_v = plsc.all_reduce_population_count(m)
            inc = plsc.cumsum(m.astype(jnp.int32))
            p = cnt_v + inc - 1
            posv = (c * _SCAN_CHUNK) + i * 16 + iot
            e = ((v - lo) << 16) | posv
            plsc.store_scatter(gpk, [p >> 7, p & (_LK - 1)], e, mask=m)
            return cnt_v + pc_v

        return lax.fori_loop(0, _SCAN_CHUNK // 16, it, cnt_v)

    cnt_v = jnp.zeros((16,), jnp.int32)
    for c in range(_NSCAN):
        cnt_v = scan_chunk(c, cnt_v)
    cnt = cnt_v[0]
    nwv = (cnt + 15) >> 4

    def walk(cb, ce, p):
        cbp = cb << 16
        cep = ce << 16
        pbase = p * _PCAP

        def it(w, kv):
            gv = gpk[w >> 3, pl.ds((w & 7) * 16, 16)]
            mw = (w * 16 + iot) < cnt_v
            m2 = mw & (gv >= cbp) & (gv < cep)
            inc = plsc.cumsum(m2.astype(jnp.int32))
            ordv = kv + inc - 1
            mp = m2 & (ordv >= pbase) & (ordv < pbase + _PCAP)
            q = ordv - pbase
            plsc.store_scatter(cpk, [q >> 7, q & (_LK - 1)], gv, mask=mp)
            plsc.store_scatter(cpos, [q >> 7, q & (_LK - 1)], gv & 0xFFFF,
                               mask=mp)
            return kv + plsc.all_reduce_population_count(m2)

        return lax.fori_loop(0, nwv, it, jnp.zeros((16,), jnp.int32))[0]

    def do_chunk(ci, z):
        cb = jnp.where(ci < (rpw >> 10), ci * _CR, rpw - _CR)
        ce = cb + _CR
        pltpu.sync_copy(mem.at[pl.ds(lo + cb, _CR)], buf)

        def pass_body(carry):
            p, _ = carry
            kc = walk(cb, ce, p)
            kp = jnp.clip(kc - p * _PCAP, 0, _PCAP)
            ngr = (kp + (_LK - 1)) >> 7

            def group(g, z2):
                gather = pltpu.make_async_copy(upd.at[cpos.at[g]], pay, s_g)
                gather.start()
                gather.wait()
                kg = jnp.clip(kp - g * _LK, 0, _LK)
                for vi in range(_LK // 16):
                    ev = cpk[g, pl.ds(vi * 16, 16)]
                    for l in range(16):
                        @pl.when(vi * 16 + l < kg)
                        def _():
                            brow = (ev[l] >> 16) - cb
                            pvec = pay[vi * 16 + l]
                            plsc.store_scatter(
                                buf, [jnp.broadcast_to(brow, (16,)), iot],
                                pvec, mask=iot < _D)
                return z2

            lax.fori_loop(0, ngr, group, 0)
            return (p + 1, kc)

        lax.while_loop(lambda c: c[0] * _PCAP < c[1], pass_body,
                       (jnp.int32(0), jnp.int32(1)))
        pltpu.sync_copy(buf, out.at[pl.ds(lo + cb, _CR)])
        return z

    nchunks = (rpw >> 10) + 1
    lax.fori_loop(0, nchunks, do_chunk, 0)


@jax.jit
def _run(mem, idx, upd):
    f = pl.kernel(
        _body,
        out_type=jax.ShapeDtypeStruct((_M, _D), jnp.float32),
        mesh=plsc.VectorSubcoreMesh(
            core_axis_name="c", subcore_axis_name="s",
            num_cores=_NC, num_subcores=_NS),
        compiler_params=pltpu.CompilerParams(
            needs_layout_passes=False, use_tc_tiling_on_sc=False),
        scratch_types=[
            pltpu.VMEM((_SCAN_CHUNK,), jnp.int32),
            pltpu.VMEM((_GROWS, _LK), jnp.int32),
            pltpu.VMEM((_PROWS, _LK), jnp.int32),
            pltpu.VMEM((_PROWS, _LK), jnp.int32),
            pltpu.VMEM((_LK, 16), jnp.float32),
            pltpu.VMEM((_CR, _D), jnp.float32),
            pltpu.SemaphoreType.DMA,
        ],
    )
    return f(mem, idx, upd)


def kernel(mem, srcID, src_feature, dstID, dst_feature):
    idx = jnp.concatenate([srcID, dstID], axis=0)
    upd = jnp.pad(jnp.concatenate([src_feature, dst_feature], axis=0),
                  ((0, 0), (0, 1)))
    return _run(mem, idx, upd)
